# baseline (speedup 1.0000x reference)
import jax
import jax.numpy as jnp
from jax.experimental import pallas as pl

D = 128
BLK = 2000


def _proj_body(x_ref, s_ref, wenc_ref, w_ref, a_ref, h_ref, l_ref):
    x = x_ref[...] + s_ref[...] * wenc_ref[...]
    h = jnp.dot(x, w_ref[...], preferred_element_type=jnp.float32)
    h_ref[...] = h
    l_ref[...] = jnp.sum(h * a_ref[...], axis=1, keepdims=True)


def _proj(x, s, wenc, w, avec):
    n = x.shape[0]
    grid = (n // BLK,)
    h, l = pl.pallas_call(
        _proj_body,
        grid=grid,
        in_specs=[
            pl.BlockSpec((BLK, D), lambda i: (i, 0)),
            pl.BlockSpec((BLK, 1), lambda i: (i, 0)),
            pl.BlockSpec((1, D), lambda i: (0, 0)),
            pl.BlockSpec((D, D), lambda i: (0, 0)),
            pl.BlockSpec((1, D), lambda i: (0, 0)),
        ],
        out_specs=[
            pl.BlockSpec((BLK, D), lambda i: (i, 0)),
            pl.BlockSpec((BLK, 1), lambda i: (i, 0)),
        ],
        out_shape=[
            jax.ShapeDtypeStruct((n, D), jnp.float32),
            jax.ShapeDtypeStruct((n, 1), jnp.float32),
        ],
    )(x, s.reshape(n, 1), wenc.reshape(1, D), w, avec.reshape(1, D))
    return h, l.reshape(n)


def _cat_body(a1_ref, a2_ref, a3_ref, x_ref, ts_ref, dg_ref,
              w2_ref, w3_ref, b1_ref, b2_ref, b3_ref,
              cw1_ref, cw2_ref, cw3_ref, cb_ref, o_ref):
    x = x_ref[...]
    u1 = a1_ref[...] + x + b1_ref[...]
    u2 = a2_ref[...] + x + ts_ref[...] * w2_ref[...] + b2_ref[...]
    u3 = a3_ref[...] + x + dg_ref[...] * w3_ref[...] + b3_ref[...]
    acc = jnp.dot(u1, cw1_ref[...], preferred_element_type=jnp.float32)
    acc += jnp.dot(u2, cw2_ref[...], preferred_element_type=jnp.float32)
    acc += jnp.dot(u3, cw3_ref[...], preferred_element_type=jnp.float32)
    o_ref[...] = acc + cb_ref[...]


def _cat(a1, a2, a3, x, ts, dg, wenc2, wenc3, b1, b2, b3, cat_w, cat_b):
    n = x.shape[0]
    row = lambda i: (i, 0)
    fix = lambda i: (0, 0)
    cw1, cw2, cw3 = cat_w[:D], cat_w[D:2 * D], cat_w[2 * D:]
    return pl.pallas_call(
        _cat_body,
        grid=(n // BLK,),
        in_specs=[
            pl.BlockSpec((BLK, D), row), pl.BlockSpec((BLK, D), row),
            pl.BlockSpec((BLK, D), row), pl.BlockSpec((BLK, D), row),
            pl.BlockSpec((BLK, 1), row), pl.BlockSpec((BLK, 1), row),
            pl.BlockSpec((1, D), fix), pl.BlockSpec((1, D), fix),
            pl.BlockSpec((1, D), fix), pl.BlockSpec((1, D), fix),
            pl.BlockSpec((1, D), fix),
            pl.BlockSpec((D, D), fix), pl.BlockSpec((D, D), fix),
            pl.BlockSpec((D, D), fix), pl.BlockSpec((1, D), fix),
        ],
        out_specs=pl.BlockSpec((BLK, D), row),
        out_shape=jax.ShapeDtypeStruct((n, D), jnp.float32),
    )(a1, a2, a3, x, ts.reshape(n, 1), dg.reshape(n, 1),
      wenc2.reshape(1, D), wenc3.reshape(1, D),
      b1.reshape(1, D), b2.reshape(1, D), b3.reshape(1, D),
      cw1, cw2, cw3, cat_b.reshape(1, D))


def _bn_silu_body(x_ref, mu_ref, inv_ref, g_ref, b_ref, o_ref):
    xn = (x_ref[...] - mu_ref[...]) * inv_ref[...] * g_ref[...] + b_ref[...]
    o_ref[...] = xn * jax.nn.sigmoid(xn)


def _bn_silu(x, g, b):
    n = x.shape[0]
    mu = x.mean(0)
    inv = jax.lax.rsqrt(x.var(0) + 1e-5)
    row = lambda i: (i, 0)
    fix = lambda i: (0, 0)
    return pl.pallas_call(
        _bn_silu_body,
        grid=(n // BLK,),
        in_specs=[
            pl.BlockSpec((BLK, D), row),
            pl.BlockSpec((1, D), fix), pl.BlockSpec((1, D), fix),
            pl.BlockSpec((1, D), fix), pl.BlockSpec((1, D), fix),
        ],
        out_specs=pl.BlockSpec((BLK, D), row),
        out_shape=jax.ShapeDtypeStruct((n, D), jnp.float32),
    )(x, mu.reshape(1, D), inv.reshape(1, D), g.reshape(1, D), b.reshape(1, D))


def _edge_agg(hs, el, er, es, ed, nd):
    e = el[es] + er[ed]
    e = jnp.where(e > 0, e, 0.01 * e)
    m = jax.ops.segment_max(e, ed, num_segments=nd)
    m = jnp.where(jnp.isfinite(m), m, 0.0)
    w = jnp.exp(e - m[ed])
    z = jax.ops.segment_sum(w, ed, num_segments=nd)
    a = w / (z[ed] + 1e-9)
    return jax.ops.segment_sum(a[:, None] * hs[es], ed, num_segments=nd)


def _layer(hu, hi, P, es, ed, ts_u, ts_i, dg_u, dg_i):
    nu, ni = hu.shape[0], hi.shape[0]
    z = jnp.zeros((D,), jnp.float32)
    zu = jnp.zeros((nu,), jnp.float32)
    zi = jnp.zeros((ni,), jnp.float32)

    p1g, p1b = P["h1_go"], P["h1_back"]
    p2g, p2b = P["h2_go"], P["h2_back"]
    p3g, p3b = P["h3_go"], P["h3_back"]

    hs1g, el1g = _proj(hu, zu, z, p1g["W"], p1g["al"])
    hd1g, er1g = _proj(hi, zi, z, p1g["W"], p1g["ar"])
    hs2g, el2g = _proj(hu, ts_u, p2g["wenc"], p2g["W"], p2g["al"])
    hd2g, er2g = _proj(hi, ts_i, p2g["wenc"], p2g["W"], p2g["ar"])
    hs3g, el3g = _proj(hu, dg_u, p3g["wenc"], p3g["W"], p3g["al"])
    hd3g, er3g = _proj(hi, dg_i, p3g["wenc"], p3g["W"], p3g["ar"])
    hs1b, el1b = _proj(hi, zi, z, p1b["W"], p1b["al"])
    hd1b, er1b = _proj(hu, zu, z, p1b["W"], p1b["ar"])
    hs2b, el2b = _proj(hi, ts_i, p2b["wenc"], p2b["W"], p2b["al"])
    hd2b, er2b = _proj(hu, ts_u, p2b["wenc"], p2b["W"], p2b["ar"])
    hs3b, el3b = _proj(hi, dg_i, p3b["wenc"], p3b["W"], p3b["al"])
    hd3b, er3b = _proj(hu, dg_u, p3b["wenc"], p3b["W"], p3b["ar"])

    i1 = _edge_agg(hs1g, el1g, er1g, es, ed, ni)
    i2 = _edge_agg(hs2g, el2g, er2g, es, ed, ni)
    i3 = _edge_agg(hs3g, el3g, er3g, es, ed, ni)
    u1 = _edge_agg(hs1b, el1b, er1b, ed, es, nu)
    u2 = _edge_agg(hs2b, el2b, er2b, ed, es, nu)
    u3 = _edge_agg(hs3b, el3b, er3b, ed, es, nu)

    uc = _cat(u1, u2, u3, hu, ts_u, dg_u, p2b["wenc"], p3b["wenc"],
              p1b["b"], p2b["b"], p3b["b"], P["cat_W"], P["cat_b"])
    ic = _cat(i1, i2, i3, hi, ts_i, dg_i, p2g["wenc"], p3g["wenc"],
              p1g["b"], p2g["b"], p3g["b"], P["cat_W"], P["cat_b"])
    uo = _bn_silu(uc, P["bn_u_g"], P["bn_u_b"])
    io = _bn_silu(ic, P["bn_i_g"], P["bn_i_b"])
    return uo, io


def kernel(params, ts_u, ts_i, dg_u, dg_i, edge_src, edge_dst):
    hu = params["emb_u"]
    hi = params["emb_i"]
    hu, hi = _layer(hu, hi, params["L1"], edge_src, edge_dst,
                    ts_u, ts_i, dg_u, dg_i)
    hu, hi = _layer(hu, hi, params["L2"], edge_src, edge_dst,
                    ts_u, ts_i, dg_u, dg_i)
    return (hu, hi)


# speedup vs baseline: 6.1754x; 6.1754x over previous
import jax
import jax.numpy as jnp
from jax.experimental import pallas as pl

D = 128
BLK = 2000


def _proj_body(x_ref, s_ref, wenc_ref, w_ref, a_ref, h_ref, l_ref):
    x = x_ref[...] + s_ref[...] * wenc_ref[...]
    h = jnp.dot(x, w_ref[...], preferred_element_type=jnp.float32)
    h_ref[...] = h
    l_ref[...] = jnp.sum(h * a_ref[...], axis=1, keepdims=True)


def _proj(x, s, wenc, w, avec):
    n = x.shape[0]
    grid = (n // BLK,)
    h, l = pl.pallas_call(
        _proj_body,
        grid=grid,
        in_specs=[
            pl.BlockSpec((BLK, D), lambda i: (i, 0)),
            pl.BlockSpec((BLK, 1), lambda i: (i, 0)),
            pl.BlockSpec((1, D), lambda i: (0, 0)),
            pl.BlockSpec((D, D), lambda i: (0, 0)),
            pl.BlockSpec((1, D), lambda i: (0, 0)),
        ],
        out_specs=[
            pl.BlockSpec((BLK, D), lambda i: (i, 0)),
            pl.BlockSpec((BLK, 1), lambda i: (i, 0)),
        ],
        out_shape=[
            jax.ShapeDtypeStruct((n, D), jnp.float32),
            jax.ShapeDtypeStruct((n, 1), jnp.float32),
        ],
    )(x, s.reshape(n, 1), wenc.reshape(1, D), w, avec.reshape(1, D))
    return h, l.reshape(n)


def _cat_body(a1_ref, a2_ref, a3_ref, x_ref, ts_ref, dg_ref,
              w2_ref, w3_ref, b1_ref, b2_ref, b3_ref,
              cw1_ref, cw2_ref, cw3_ref, cb_ref, o_ref):
    x = x_ref[...]
    u1 = a1_ref[...] + x + b1_ref[...]
    u2 = a2_ref[...] + x + ts_ref[...] * w2_ref[...] + b2_ref[...]
    u3 = a3_ref[...] + x + dg_ref[...] * w3_ref[...] + b3_ref[...]
    acc = jnp.dot(u1, cw1_ref[...], preferred_element_type=jnp.float32)
    acc += jnp.dot(u2, cw2_ref[...], preferred_element_type=jnp.float32)
    acc += jnp.dot(u3, cw3_ref[...], preferred_element_type=jnp.float32)
    o_ref[...] = acc + cb_ref[...]


def _cat(a1, a2, a3, x, ts, dg, wenc2, wenc3, b1, b2, b3, cat_w, cat_b):
    n = x.shape[0]
    row = lambda i: (i, 0)
    fix = lambda i: (0, 0)
    cw1, cw2, cw3 = cat_w[:D], cat_w[D:2 * D], cat_w[2 * D:]
    return pl.pallas_call(
        _cat_body,
        grid=(n // BLK,),
        in_specs=[
            pl.BlockSpec((BLK, D), row), pl.BlockSpec((BLK, D), row),
            pl.BlockSpec((BLK, D), row), pl.BlockSpec((BLK, D), row),
            pl.BlockSpec((BLK, 1), row), pl.BlockSpec((BLK, 1), row),
            pl.BlockSpec((1, D), fix), pl.BlockSpec((1, D), fix),
            pl.BlockSpec((1, D), fix), pl.BlockSpec((1, D), fix),
            pl.BlockSpec((1, D), fix),
            pl.BlockSpec((D, D), fix), pl.BlockSpec((D, D), fix),
            pl.BlockSpec((D, D), fix), pl.BlockSpec((1, D), fix),
        ],
        out_specs=pl.BlockSpec((BLK, D), row),
        out_shape=jax.ShapeDtypeStruct((n, D), jnp.float32),
    )(a1, a2, a3, x, ts.reshape(n, 1), dg.reshape(n, 1),
      wenc2.reshape(1, D), wenc3.reshape(1, D),
      b1.reshape(1, D), b2.reshape(1, D), b3.reshape(1, D),
      cw1, cw2, cw3, cat_b.reshape(1, D))


def _bn_silu_body(x_ref, mu_ref, inv_ref, g_ref, b_ref, o_ref):
    xn = (x_ref[...] - mu_ref[...]) * inv_ref[...] * g_ref[...] + b_ref[...]
    o_ref[...] = xn * jax.nn.sigmoid(xn)


def _bn_silu(x, g, b):
    n = x.shape[0]
    mu = x.mean(0)
    inv = jax.lax.rsqrt(x.var(0) + 1e-5)
    row = lambda i: (i, 0)
    fix = lambda i: (0, 0)
    return pl.pallas_call(
        _bn_silu_body,
        grid=(n // BLK,),
        in_specs=[
            pl.BlockSpec((BLK, D), row),
            pl.BlockSpec((1, D), fix), pl.BlockSpec((1, D), fix),
            pl.BlockSpec((1, D), fix), pl.BlockSpec((1, D), fix),
        ],
        out_specs=pl.BlockSpec((BLK, D), row),
        out_shape=jax.ShapeDtypeStruct((n, D), jnp.float32),
    )(x, mu.reshape(1, D), inv.reshape(1, D), g.reshape(1, D), b.reshape(1, D))


def _edge_agg3(hs_list, el_list, er_list, es, ed, nd):
    el = jnp.stack(el_list, axis=1)
    er = jnp.stack(er_list, axis=1)
    e = el[es] + er[ed]
    e = jnp.where(e > 0, e, 0.01 * e)
    m = jax.ops.segment_max(e, ed, num_segments=nd)
    m = jnp.where(jnp.isfinite(m), m, 0.0)
    w = jnp.exp(e - m[ed])
    z = jax.ops.segment_sum(w, ed, num_segments=nd)
    a = w / (z[ed] + 1e-9)
    hs_cat = jnp.concatenate(hs_list, axis=1)
    g = hs_cat[es]
    ecnt = es.shape[0]
    msg = (g.reshape(ecnt, 3, D) * a[:, :, None]).reshape(ecnt, 3 * D)
    out = jax.ops.segment_sum(msg, ed, num_segments=nd)
    return out[:, :D], out[:, D:2 * D], out[:, 2 * D:]


def _layer(hu, hi, P, es, ed, ts_u, ts_i, dg_u, dg_i):
    nu, ni = hu.shape[0], hi.shape[0]
    z = jnp.zeros((D,), jnp.float32)
    zu = jnp.zeros((nu,), jnp.float32)
    zi = jnp.zeros((ni,), jnp.float32)

    p1g, p1b = P["h1_go"], P["h1_back"]
    p2g, p2b = P["h2_go"], P["h2_back"]
    p3g, p3b = P["h3_go"], P["h3_back"]

    hs1g, el1g = _proj(hu, zu, z, p1g["W"], p1g["al"])
    hd1g, er1g = _proj(hi, zi, z, p1g["W"], p1g["ar"])
    hs2g, el2g = _proj(hu, ts_u, p2g["wenc"], p2g["W"], p2g["al"])
    hd2g, er2g = _proj(hi, ts_i, p2g["wenc"], p2g["W"], p2g["ar"])
    hs3g, el3g = _proj(hu, dg_u, p3g["wenc"], p3g["W"], p3g["al"])
    hd3g, er3g = _proj(hi, dg_i, p3g["wenc"], p3g["W"], p3g["ar"])
    hs1b, el1b = _proj(hi, zi, z, p1b["W"], p1b["al"])
    hd1b, er1b = _proj(hu, zu, z, p1b["W"], p1b["ar"])
    hs2b, el2b = _proj(hi, ts_i, p2b["wenc"], p2b["W"], p2b["al"])
    hd2b, er2b = _proj(hu, ts_u, p2b["wenc"], p2b["W"], p2b["ar"])
    hs3b, el3b = _proj(hi, dg_i, p3b["wenc"], p3b["W"], p3b["al"])
    hd3b, er3b = _proj(hu, dg_u, p3b["wenc"], p3b["W"], p3b["ar"])

    i1, i2, i3 = _edge_agg3([hs1g, hs2g, hs3g], [el1g, el2g, el3g],
                            [er1g, er2g, er3g], es, ed, ni)
    u1, u2, u3 = _edge_agg3([hs1b, hs2b, hs3b], [el1b, el2b, el3b],
                            [er1b, er2b, er3b], ed, es, nu)

    uc = _cat(u1, u2, u3, hu, ts_u, dg_u, p2b["wenc"], p3b["wenc"],
              p1b["b"], p2b["b"], p3b["b"], P["cat_W"], P["cat_b"])
    ic = _cat(i1, i2, i3, hi, ts_i, dg_i, p2g["wenc"], p3g["wenc"],
              p1g["b"], p2g["b"], p3g["b"], P["cat_W"], P["cat_b"])
    uo = _bn_silu(uc, P["bn_u_g"], P["bn_u_b"])
    io = _bn_silu(ic, P["bn_i_g"], P["bn_i_b"])
    return uo, io


def kernel(params, ts_u, ts_i, dg_u, dg_i, edge_src, edge_dst):
    hu = params["emb_u"]
    hi = params["emb_i"]
    hu, hi = _layer(hu, hi, params["L1"], edge_src, edge_dst,
                    ts_u, ts_i, dg_u, dg_i)
    hu, hi = _layer(hu, hi, params["L2"], edge_src, edge_dst,
                    ts_u, ts_i, dg_u, dg_i)
    return (hu, hi)


# speedup vs baseline: 6.1954x; 1.0032x over previous
import jax
import jax.numpy as jnp
from jax.experimental import pallas as pl

D = 128
BLK = 2000


def _proj_body(x_ref, s_ref, wenc_ref, w_ref, a_ref, h_ref, l_ref):
    x = x_ref[...] + s_ref[...] * wenc_ref[...]
    h = jnp.dot(x, w_ref[...], preferred_element_type=jnp.float32)
    h_ref[...] = h
    l_ref[...] = jnp.sum(h * a_ref[...], axis=1, keepdims=True)


def _proj(x, s, wenc, w, avec):
    n = x.shape[0]
    grid = (n // BLK,)
    h, l = pl.pallas_call(
        _proj_body,
        grid=grid,
        in_specs=[
            pl.BlockSpec((BLK, D), lambda i: (i, 0)),
            pl.BlockSpec((BLK, 1), lambda i: (i, 0)),
            pl.BlockSpec((1, D), lambda i: (0, 0)),
            pl.BlockSpec((D, D), lambda i: (0, 0)),
            pl.BlockSpec((1, D), lambda i: (0, 0)),
        ],
        out_specs=[
            pl.BlockSpec((BLK, D), lambda i: (i, 0)),
            pl.BlockSpec((BLK, 1), lambda i: (i, 0)),
        ],
        out_shape=[
            jax.ShapeDtypeStruct((n, D), jnp.float32),
            jax.ShapeDtypeStruct((n, 1), jnp.float32),
        ],
    )(x, s.reshape(n, 1), wenc.reshape(1, D), w, avec.reshape(1, D))
    return h, l.reshape(n)


def _cat_body(a1_ref, a2_ref, a3_ref, x_ref, ts_ref, dg_ref,
              w2_ref, w3_ref, b1_ref, b2_ref, b3_ref,
              cw1_ref, cw2_ref, cw3_ref, cb_ref, o_ref):
    x = x_ref[...]
    u1 = a1_ref[...] + x + b1_ref[...]
    u2 = a2_ref[...] + x + ts_ref[...] * w2_ref[...] + b2_ref[...]
    u3 = a3_ref[...] + x + dg_ref[...] * w3_ref[...] + b3_ref[...]
    acc = jnp.dot(u1, cw1_ref[...], preferred_element_type=jnp.float32)
    acc += jnp.dot(u2, cw2_ref[...], preferred_element_type=jnp.float32)
    acc += jnp.dot(u3, cw3_ref[...], preferred_element_type=jnp.float32)
    o_ref[...] = acc + cb_ref[...]


def _cat(a1, a2, a3, x, ts, dg, wenc2, wenc3, b1, b2, b3, cat_w, cat_b):
    n = x.shape[0]
    row = lambda i: (i, 0)
    fix = lambda i: (0, 0)
    cw1, cw2, cw3 = cat_w[:D], cat_w[D:2 * D], cat_w[2 * D:]
    return pl.pallas_call(
        _cat_body,
        grid=(n // BLK,),
        in_specs=[
            pl.BlockSpec((BLK, D), row), pl.BlockSpec((BLK, D), row),
            pl.BlockSpec((BLK, D), row), pl.BlockSpec((BLK, D), row),
            pl.BlockSpec((BLK, 1), row), pl.BlockSpec((BLK, 1), row),
            pl.BlockSpec((1, D), fix), pl.BlockSpec((1, D), fix),
            pl.BlockSpec((1, D), fix), pl.BlockSpec((1, D), fix),
            pl.BlockSpec((1, D), fix),
            pl.BlockSpec((D, D), fix), pl.BlockSpec((D, D), fix),
            pl.BlockSpec((D, D), fix), pl.BlockSpec((1, D), fix),
        ],
        out_specs=pl.BlockSpec((BLK, D), row),
        out_shape=jax.ShapeDtypeStruct((n, D), jnp.float32),
    )(a1, a2, a3, x, ts.reshape(n, 1), dg.reshape(n, 1),
      wenc2.reshape(1, D), wenc3.reshape(1, D),
      b1.reshape(1, D), b2.reshape(1, D), b3.reshape(1, D),
      cw1, cw2, cw3, cat_b.reshape(1, D))


def _bn_silu_body(x_ref, mu_ref, inv_ref, g_ref, b_ref, o_ref):
    xn = (x_ref[...] - mu_ref[...]) * inv_ref[...] * g_ref[...] + b_ref[...]
    o_ref[...] = xn * jax.nn.sigmoid(xn)


def _bn_silu(x, g, b):
    n = x.shape[0]
    mu = x.mean(0)
    inv = jax.lax.rsqrt(x.var(0) + 1e-5)
    row = lambda i: (i, 0)
    fix = lambda i: (0, 0)
    return pl.pallas_call(
        _bn_silu_body,
        grid=(n // BLK,),
        in_specs=[
            pl.BlockSpec((BLK, D), row),
            pl.BlockSpec((1, D), fix), pl.BlockSpec((1, D), fix),
            pl.BlockSpec((1, D), fix), pl.BlockSpec((1, D), fix),
        ],
        out_specs=pl.BlockSpec((BLK, D), row),
        out_shape=jax.ShapeDtypeStruct((n, D), jnp.float32),
    )(x, mu.reshape(1, D), inv.reshape(1, D), g.reshape(1, D), b.reshape(1, D))


def _edge_agg3(hs_list, el_list, er_list, es, ed, nd):
    el = jnp.stack(el_list, axis=1)
    er = jnp.stack(er_list, axis=1)
    e = el[es] + er[ed]
    e = jnp.where(e > 0, e, 0.01 * e)
    m = jax.ops.segment_max(e, ed, num_segments=nd, indices_are_sorted=True)
    m = jnp.where(jnp.isfinite(m), m, 0.0)
    w = jnp.exp(e - m[ed])
    z = jax.ops.segment_sum(w, ed, num_segments=nd, indices_are_sorted=True)
    a = w / (z[ed] + 1e-9)
    hs_cat = jnp.concatenate(hs_list, axis=1)
    g = hs_cat[es]
    ecnt = es.shape[0]
    msg = (g.reshape(ecnt, 3, D) * a[:, :, None]).reshape(ecnt, 3 * D)
    out = jax.ops.segment_sum(msg, ed, num_segments=nd,
                              indices_are_sorted=True)
    return out[:, :D], out[:, D:2 * D], out[:, 2 * D:]


def _layer(hu, hi, P, go_edges, back_edges, ts_u, ts_i, dg_u, dg_i):
    es_g, ed_g = go_edges
    es_b, ed_b = back_edges
    nu, ni = hu.shape[0], hi.shape[0]
    z = jnp.zeros((D,), jnp.float32)
    zu = jnp.zeros((nu,), jnp.float32)
    zi = jnp.zeros((ni,), jnp.float32)

    p1g, p1b = P["h1_go"], P["h1_back"]
    p2g, p2b = P["h2_go"], P["h2_back"]
    p3g, p3b = P["h3_go"], P["h3_back"]

    hs1g, el1g = _proj(hu, zu, z, p1g["W"], p1g["al"])
    hd1g, er1g = _proj(hi, zi, z, p1g["W"], p1g["ar"])
    hs2g, el2g = _proj(hu, ts_u, p2g["wenc"], p2g["W"], p2g["al"])
    hd2g, er2g = _proj(hi, ts_i, p2g["wenc"], p2g["W"], p2g["ar"])
    hs3g, el3g = _proj(hu, dg_u, p3g["wenc"], p3g["W"], p3g["al"])
    hd3g, er3g = _proj(hi, dg_i, p3g["wenc"], p3g["W"], p3g["ar"])
    hs1b, el1b = _proj(hi, zi, z, p1b["W"], p1b["al"])
    hd1b, er1b = _proj(hu, zu, z, p1b["W"], p1b["ar"])
    hs2b, el2b = _proj(hi, ts_i, p2b["wenc"], p2b["W"], p2b["al"])
    hd2b, er2b = _proj(hu, ts_u, p2b["wenc"], p2b["W"], p2b["ar"])
    hs3b, el3b = _proj(hi, dg_i, p3b["wenc"], p3b["W"], p3b["al"])
    hd3b, er3b = _proj(hu, dg_u, p3b["wenc"], p3b["W"], p3b["ar"])

    i1, i2, i3 = _edge_agg3([hs1g, hs2g, hs3g], [el1g, el2g, el3g],
                            [er1g, er2g, er3g], es_g, ed_g, ni)
    u1, u2, u3 = _edge_agg3([hs1b, hs2b, hs3b], [el1b, el2b, el3b],
                            [er1b, er2b, er3b], es_b, ed_b, nu)

    uc = _cat(u1, u2, u3, hu, ts_u, dg_u, p2b["wenc"], p3b["wenc"],
              p1b["b"], p2b["b"], p3b["b"], P["cat_W"], P["cat_b"])
    ic = _cat(i1, i2, i3, hi, ts_i, dg_i, p2g["wenc"], p3g["wenc"],
              p1g["b"], p2g["b"], p3g["b"], P["cat_W"], P["cat_b"])
    uo = _bn_silu(uc, P["bn_u_g"], P["bn_u_b"])
    io = _bn_silu(ic, P["bn_i_g"], P["bn_i_b"])
    return uo, io


def kernel(params, ts_u, ts_i, dg_u, dg_i, edge_src, edge_dst):
    ord_d = jnp.argsort(edge_dst)
    go_edges = (edge_src[ord_d], edge_dst[ord_d])
    ord_s = jnp.argsort(edge_src)
    back_edges = (edge_dst[ord_s], edge_src[ord_s])
    hu = params["emb_u"]
    hi = params["emb_i"]
    hu, hi = _layer(hu, hi, params["L1"], go_edges, back_edges,
                    ts_u, ts_i, dg_u, dg_i)
    hu, hi = _layer(hu, hi, params["L2"], go_edges, back_edges,
                    ts_u, ts_i, dg_u, dg_i)
    return (hu, hi)
